# Initial kernel scaffold; baseline (speedup 1.0000x reference)
import jax
import jax.numpy as jnp
from jax.experimental import pallas as pl


def kernel(input, embed):
    raise NotImplementedError("write your pallas kernel here")


# speedup vs baseline: 3.3458x; 3.3458x over previous
import functools

import jax
import jax.numpy as jnp
from jax import lax
from jax.experimental import pallas as pl
from jax.experimental.pallas import tpu as pltpu
from jax.experimental.pallas import tpu_sc as plsc

_DIM = 32
_K = 8192
_N = 4608
_BN = 576
_KT = 2048

_NW = 32
_RPW = _N // _NW
_C = 72


def _tc_argmin_body(x_ref, e_ref, idx_ref, dsum_ref):
    x = x_ref[...]
    xnorm = jnp.sum(x * x, axis=1, keepdims=True)
    fiota = lax.broadcasted_iota(jnp.int32, (_BN, _KT), 1).astype(jnp.float32)
    best_val = jnp.full((_BN,), jnp.inf, jnp.float32)
    best_fidx = jnp.zeros((_BN,), jnp.float32)
    for t in range(_K // _KT):
        e = e_ref[:, t * _KT:(t + 1) * _KT]
        scores = jnp.dot(x, e, preferred_element_type=jnp.float32)
        enorm = jnp.sum(e * e, axis=0, keepdims=True)
        dist = xnorm - 2.0 * scores + enorm
        lmin = jnp.min(dist, axis=1)
        larg = jnp.min(jnp.where(dist == lmin[:, None], fiota, jnp.inf), axis=1)
        better = lmin < best_val
        best_val = jnp.where(better, lmin, best_val)
        best_fidx = jnp.where(better, larg + jnp.float32(t * _KT), best_fidx)
    idx_ref[0, 0, :] = best_fidx.astype(jnp.int32)
    dsum_ref[0, 0, :] = jnp.sum(best_val, keepdims=True)


def _argmin_codes(flat, embed):
    nblk = flat.shape[0] // _BN
    return pl.pallas_call(
        _tc_argmin_body,
        grid=(nblk,),
        in_specs=[
            pl.BlockSpec((_BN, _DIM), lambda i: (i, 0)),
            pl.BlockSpec((_DIM, _K), lambda i: (0, 0)),
        ],
        out_specs=[
            pl.BlockSpec((1, 1, _BN), lambda i: (i, 0, 0)),
            pl.BlockSpec((1, 1, 1), lambda i: (i, 0, 0)),
        ],
        out_shape=[
            jax.ShapeDtypeStruct((nblk, 1, _BN), jnp.int32),
            jax.ShapeDtypeStruct((nblk, 1, 1), jnp.float32),
        ],
    )(flat, embed)


@functools.cache
def _sc_gather():
    @functools.partial(
        pl.kernel,
        mesh=plsc.VectorSubcoreMesh(core_axis_name="c", subcore_axis_name="s"),
        out_type=jax.ShapeDtypeStruct((_N, _DIM), jnp.float32),
        scratch_types=[
            pltpu.VMEM((_C,), jnp.int32),
            pltpu.VMEM((_C, _DIM), jnp.float32),
            pltpu.SemaphoreType.DMA,
        ],
        compiler_params=pltpu.CompilerParams(use_tc_tiling_on_sc=False),
    )
    def sc_body(emb_t, idx, q_out, idx_v, rows_v, sem):
        wid = lax.axis_index("s") * 2 + lax.axis_index("c")
        for j in range(_RPW // _C):
            base = wid * _RPW + j * _C
            pltpu.sync_copy(idx.at[pl.ds(base, _C)], idx_v)
            pltpu.async_copy(emb_t.at[idx_v], rows_v, sem).wait()
            pltpu.sync_copy(rows_v, q_out.at[pl.ds(base, _C)])

    return sc_body


def kernel(input, embed):
    flat = input.reshape(-1, _DIM)
    idx3, dsum = _argmin_codes(flat, embed)
    emb_t = embed.T
    q_flat = _sc_gather()(emb_t, idx3.reshape(-1))
    quantize = q_flat.reshape(input.shape)
    embed_ind = idx3.reshape(input.shape[:-1])
    diff = jnp.sum(dsum) / jnp.float32(flat.size)
    return (quantize, embed_ind, diff, diff)


# speedup vs baseline: 3.4289x; 1.0248x over previous
import functools

import jax
import jax.numpy as jnp
from jax import lax
from jax.experimental import pallas as pl
from jax.experimental.pallas import tpu as pltpu
from jax.experimental.pallas import tpu_sc as plsc

_DIM = 32
_K = 8192
_N = 4608
_BN = 576
_NB = _N // _BN
_KT = 4096

_NW = 32
_RPW = _N // _NW
_C = 72

_INV_COUNT = 1.0 / float(_N * _DIM)


def _tc_argmin_body(x_ref, e_ref, idx_ref, dsum_ref):
    pid = pl.program_id(0)
    x = x_ref[0]
    x2 = x + x
    xnorm = jnp.sum(x * x, axis=1, keepdims=True)
    fiota = lax.broadcasted_iota(jnp.int32, (_BN, _KT), 1).astype(jnp.float32)
    best_val = jnp.full((_BN,), jnp.inf, jnp.float32)
    best_fidx = jnp.zeros((_BN,), jnp.float32)
    for t in range(_K // _KT):
        e = e_ref[:, t * _KT:(t + 1) * _KT]
        scores2 = jnp.dot(x2, e, preferred_element_type=jnp.float32)
        enorm = jnp.sum(e * e, axis=0, keepdims=True)
        dist = (xnorm - scores2) + enorm
        lmin = jnp.min(dist, axis=1)
        larg = jnp.min(jnp.where(dist == lmin[:, None], fiota, jnp.inf), axis=1)
        better = lmin < best_val
        best_val = jnp.where(better, lmin, best_val)
        best_fidx = jnp.where(better, larg + jnp.float32(t * _KT), best_fidx)
    idx_ref[pid, :] = best_fidx.astype(jnp.int32)
    bsum = jnp.sum(best_val)[None, None] * jnp.float32(_INV_COUNT)

    @pl.when(pid == 0)
    def _():
        dsum_ref[...] = jnp.zeros((1, 1), jnp.float32)

    dsum_ref[...] += bsum


def _argmin_codes(flat, embed):
    return pl.pallas_call(
        _tc_argmin_body,
        grid=(_NB,),
        in_specs=[
            pl.BlockSpec((1, _BN, _DIM), lambda i: (i, 0, 0)),
            pl.BlockSpec((_DIM, _K), lambda i: (0, 0)),
        ],
        out_specs=[
            pl.BlockSpec((_NB, _BN), lambda i: (0, 0)),
            pl.BlockSpec((1, 1), lambda i: (0, 0)),
        ],
        out_shape=[
            jax.ShapeDtypeStruct((_NB, _BN), jnp.int32),
            jax.ShapeDtypeStruct((1, 1), jnp.float32),
        ],
    )(flat, embed)


@functools.cache
def _sc_gather():
    @functools.partial(
        pl.kernel,
        mesh=plsc.VectorSubcoreMesh(core_axis_name="c", subcore_axis_name="s"),
        out_type=jax.ShapeDtypeStruct((_N, _DIM), jnp.float32),
        scratch_types=[
            pltpu.VMEM((_C,), jnp.int32),
            pltpu.VMEM((_C, _DIM), jnp.float32),
            pltpu.SemaphoreType.DMA,
        ],
        compiler_params=pltpu.CompilerParams(use_tc_tiling_on_sc=False),
    )
    def sc_body(emb_t, idx, q_out, idx_v, rows_v, sem):
        wid = lax.axis_index("s") * 2 + lax.axis_index("c")
        for j in range(_RPW // _C):
            base = wid * _RPW + j * _C
            pltpu.sync_copy(idx.at[pl.ds(base, _C)], idx_v)
            pltpu.async_copy(emb_t.at[idx_v], rows_v, sem).wait()
            pltpu.sync_copy(rows_v, q_out.at[pl.ds(base, _C)])

    return sc_body


def kernel(input, embed):
    idx, dsum = _argmin_codes(input, embed)
    emb_t = embed.T
    q_flat = _sc_gather()(emb_t, idx.reshape(-1))
    quantize = q_flat.reshape(input.shape)
    embed_ind = idx
    diff = dsum.reshape(())
    return (quantize, embed_ind, diff, diff)


# speedup vs baseline: 3.7814x; 1.1028x over previous
import functools

import jax
import jax.numpy as jnp
from jax import lax
from jax.experimental import pallas as pl
from jax.experimental.pallas import tpu as pltpu
from jax.experimental.pallas import tpu_sc as plsc

_DIM = 32
_K = 8192
_N = 4608
_BN = 576
_NB = _N // _BN
_KT = 4096

_NW = 32
_RPW = _N // _NW
_C = 72

_INV_COUNT = 1.0 / float(_N * _DIM)


def _tc_argmin_body(x_ref, e_ref, idx_ref, dsum_ref):
    pid = pl.program_id(0)
    x = x_ref[0]
    x2 = x + x
    xnorm = jnp.sum(x * x, axis=1, keepdims=True)
    e = e_ref[...]
    scores2 = jnp.dot(x2, e, preferred_element_type=jnp.float32)
    enorm = jnp.sum(e * e, axis=0, keepdims=True)
    ng = _K // 128
    m = (xnorm - scores2[:, :128]) + enorm[:, :128]
    gi = jnp.zeros((_BN, 128), jnp.float32)
    for g in range(1, ng):
        dg = (xnorm - scores2[:, g * 128:(g + 1) * 128]) + enorm[:, g * 128:(g + 1) * 128]
        lt = dg < m
        m = jnp.minimum(m, dg)
        gi = jnp.where(lt, jnp.float32(g), gi)
    gmin = jnp.min(m, axis=1)
    lidx = lax.broadcasted_iota(jnp.int32, (_BN, 128), 1).astype(jnp.float32)
    fidx = gi * 128.0 + lidx
    fbest = jnp.min(jnp.where(m == gmin[:, None], fidx, jnp.inf), axis=1)
    idx_ref[pid, :] = fbest.astype(jnp.int32)
    bsum = jnp.sum(gmin)[None, None] * jnp.float32(_INV_COUNT)

    @pl.when(pid == 0)
    def _():
        dsum_ref[...] = jnp.zeros((1, 1), jnp.float32)

    dsum_ref[...] += bsum


def _argmin_codes(flat, embed):
    return pl.pallas_call(
        _tc_argmin_body,
        grid=(_NB,),
        in_specs=[
            pl.BlockSpec((1, _BN, _DIM), lambda i: (i, 0, 0)),
            pl.BlockSpec((_DIM, _K), lambda i: (0, 0)),
        ],
        out_specs=[
            pl.BlockSpec((_NB, _BN), lambda i: (0, 0)),
            pl.BlockSpec((1, 1), lambda i: (0, 0)),
        ],
        out_shape=[
            jax.ShapeDtypeStruct((_NB, _BN), jnp.int32),
            jax.ShapeDtypeStruct((1, 1), jnp.float32),
        ],
    )(flat, embed)


@functools.cache
def _sc_gather():
    @functools.partial(
        pl.kernel,
        mesh=plsc.VectorSubcoreMesh(core_axis_name="c", subcore_axis_name="s"),
        out_type=jax.ShapeDtypeStruct((_N, _DIM), jnp.float32),
        scratch_types=[
            pltpu.VMEM((_C,), jnp.int32),
            pltpu.VMEM((_C, _DIM), jnp.float32),
            pltpu.SemaphoreType.DMA,
        ],
        compiler_params=pltpu.CompilerParams(use_tc_tiling_on_sc=False),
    )
    def sc_body(emb_t, idx, q_out, idx_v, rows_v, sem):
        wid = lax.axis_index("s") * 2 + lax.axis_index("c")
        for j in range(_RPW // _C):
            base = wid * _RPW + j * _C
            pltpu.sync_copy(idx.at[pl.ds(base, _C)], idx_v)
            pltpu.async_copy(emb_t.at[idx_v], rows_v, sem).wait()
            pltpu.sync_copy(rows_v, q_out.at[pl.ds(base, _C)])

    return sc_body


def kernel(input, embed):
    idx, dsum = _argmin_codes(input, embed)
    emb_t = embed.T
    q_flat = _sc_gather()(emb_t, idx.reshape(-1))
    quantize = q_flat.reshape(input.shape)
    embed_ind = idx
    diff = dsum.reshape(())
    return (quantize, embed_ind, diff, diff)
